# baseline (speedup 1.0000x reference)
import functools

import jax
import jax.numpy as jnp
from jax import lax
from jax.experimental import pallas as pl
from jax.experimental.pallas import tpu as pltpu
from jax.experimental.pallas import tpu_sc as plsc

B = 16
D = 64
CACHE_SIZE = 1000000
NUM_CLASSES = 10
THRESHOLD = 1.0

NBLK = 125
BK = CACHE_SIZE // NBLK


def _tc_body(x_ref, keys_ref, w_ref, b_ref, val_ref, idx_ref, logits_ref):
    i = pl.program_id(0)
    x = x_ref[...]
    k = keys_ref[...]
    xsq = jnp.sum(x * x, axis=1, keepdims=True)
    ksq = jnp.sum(k * k, axis=1)
    xk = lax.dot_general(x, k, (((1,), (1,)), ((), ())),
                         preferred_element_type=jnp.float32,
                         precision=lax.Precision.HIGHEST)
    d2 = jnp.maximum(xsq + ksq[None, :] - 2.0 * xk, 0.0)
    val_ref[...] = jnp.min(d2, axis=1).reshape(1, 1, B)
    idx_ref[...] = (jnp.argmin(d2, axis=1).astype(jnp.int32)
                    + i * BK).reshape(1, 1, B)

    @pl.when(i == 0)
    def _():
        lt = lax.dot_general(w_ref[...], x, (((0,), (1,)), ((), ())),
                             preferred_element_type=jnp.float32,
                             precision=lax.Precision.HIGHEST)
        lt = lt + b_ref[...][:, None]
        logits_ref[...] = jnp.concatenate(
            [lt, jnp.zeros((B - NUM_CLASSES, B), jnp.float32)], axis=0)


def _tc_call(x, cache_keys, W, b):
    return pl.pallas_call(
        _tc_body,
        grid=(NBLK,),
        in_specs=[
            pl.BlockSpec((B, D), lambda i: (0, 0)),
            pl.BlockSpec((BK, D), lambda i: (i, 0)),
            pl.BlockSpec((D, NUM_CLASSES), lambda i: (0, 0)),
            pl.BlockSpec((NUM_CLASSES,), lambda i: (0,)),
        ],
        out_specs=[
            pl.BlockSpec((1, 1, B), lambda i: (i, 0, 0)),
            pl.BlockSpec((1, 1, B), lambda i: (i, 0, 0)),
            pl.BlockSpec((B, B), lambda i: (0, 0)),
        ],
        out_shape=[
            jax.ShapeDtypeStruct((NBLK, 1, B), jnp.float32),
            jax.ShapeDtypeStruct((NBLK, 1, B), jnp.int32),
            jax.ShapeDtypeStruct((B, B), jnp.float32),
        ],
        compiler_params=pltpu.CompilerParams(
            dimension_semantics=("arbitrary",)),
    )(x, cache_keys, W, b)


def _sc_kernel_fn(cand_val_hbm, cand_idx_hbm, logits_hbm, preds_hbm,
                  probs_hbm, cache_hbm,
                  cv, ci, ml, idx_b, cols, pv, icv, sem):
    cid = lax.axis_index("c")
    sid = lax.axis_index("s")

    @pl.when(jnp.logical_and(cid == 0, sid == 0))
    def _():
        pltpu.sync_copy(cand_val_hbm, cv)
        pltpu.sync_copy(cand_idx_hbm, ci)
        pltpu.sync_copy(logits_hbm, ml)
        best_v = cv[0, :]
        best_i = ci[0, :]
        for r in range(1, NBLK):
            v = cv[r, :]
            m = v < best_v
            best_v = jnp.where(m, v, best_v)
            best_i = jnp.where(m, ci[r, :], best_i)
        base = best_i * NUM_CLASSES
        for c in range(NUM_CLASSES):
            idx_b[c, :] = base + c
        copies = [
            pltpu.async_copy(preds_hbm.at[idx_b.at[c]], cols.at[c], sem)
            for c in range(NUM_CLASSES)
        ]
        for cp in copies:
            cp.wait()
        is_cache = best_v <= jnp.float32(THRESHOLD * THRESHOLD)
        sel = [jnp.where(is_cache, cols[c, :], ml[c, :])
               for c in range(NUM_CLASSES)]
        mx = sel[0]
        for c in range(1, NUM_CLASSES):
            mx = jnp.maximum(mx, sel[c])
        es = [jnp.exp(s - mx) for s in sel]
        tot = es[0]
        for c in range(1, NUM_CLASSES):
            tot = tot + es[c]
        inv = jnp.float32(1.0) / tot
        for c in range(NUM_CLASSES):
            pv[c, :] = es[c] * inv
        for c in range(NUM_CLASSES, B):
            pv[c, :] = jnp.zeros((B,), jnp.float32)
        icv[...] = jnp.where(is_cache, jnp.int32(1), jnp.int32(0))
        pltpu.sync_copy(pv, probs_hbm)
        pltpu.sync_copy(icv, cache_hbm)


def _sc_call(cand_val, cand_idx, logits_t, preds_flat):
    mesh = plsc.VectorSubcoreMesh(core_axis_name="c", subcore_axis_name="s")
    f = functools.partial(
        pl.kernel, mesh=mesh,
        out_type=[
            jax.ShapeDtypeStruct((B, B), jnp.float32),
            jax.ShapeDtypeStruct((B,), jnp.int32),
        ],
        scratch_types=[
            pltpu.VMEM((NBLK, B), jnp.float32),
            pltpu.VMEM((NBLK, B), jnp.int32),
            pltpu.VMEM((B, B), jnp.float32),
            pltpu.VMEM((NUM_CLASSES, B), jnp.int32),
            pltpu.VMEM((NUM_CLASSES, B), jnp.float32),
            pltpu.VMEM((B, B), jnp.float32),
            pltpu.VMEM((B,), jnp.int32),
            pltpu.SemaphoreType.DMA,
        ],
    )(_sc_kernel_fn)
    return f(cand_val, cand_idx, logits_t, preds_flat)


def kernel(x, cache_keys, cache_preds, W, b):
    cand_val, cand_idx, logits_t = _tc_call(x, cache_keys, W, b)
    probs_t, is_cache_i32 = _sc_call(
        cand_val.reshape(NBLK, B), cand_idx.reshape(NBLK, B),
        logits_t, cache_preds.reshape(-1))
    probs = probs_t[:NUM_CLASSES, :].T
    return probs, is_cache_i32.astype(bool)


# speedup vs baseline: 1.5350x; 1.5350x over previous
import functools

import jax
import jax.numpy as jnp
from jax import lax
from jax.experimental import pallas as pl
from jax.experimental.pallas import tpu as pltpu
from jax.experimental.pallas import tpu_sc as plsc

B = 16
D = 64
CACHE_SIZE = 1000000
NUM_CLASSES = 10
THRESHOLD = 1.0

NBLK = 125
BK = CACHE_SIZE // NBLK
IDX_BITS = 13
IDX_MASK = (1 << IDX_BITS) - 1


def _tc_body(x_ref, keys_ref, w_ref, b_ref, cand_ref, logits_ref):
    x = x_ref[...]
    k = keys_ref[...]
    xsq = jnp.sum(x * x, axis=1, keepdims=True)
    ka = jnp.concatenate([k, k * k], axis=1)
    xa = jnp.concatenate(
        [-2.0 * x, jnp.ones((B, D), jnp.float32)], axis=1)
    s = lax.dot_general(xa.astype(jnp.bfloat16), ka.astype(jnp.bfloat16),
                        (((1,), (1,)), ((), ())),
                        preferred_element_type=jnp.float32)
    d2 = jnp.maximum(s + xsq, 0.0)
    bits = lax.bitcast_convert_type(d2, jnp.int32)
    lane = lax.broadcasted_iota(jnp.int32, (B, BK), 1)
    packed = (bits & ~IDX_MASK) | lane
    cand_ref[...] = jnp.min(packed, axis=1).reshape(1, 1, B)

    @pl.when(pl.program_id(0) == 0)
    def _():
        ml = jnp.dot(x, w_ref[...], preferred_element_type=jnp.float32,
                     precision=lax.Precision.HIGHEST) + b_ref[...][None, :]
        logits_ref[...] = jnp.concatenate(
            [ml, jnp.zeros((B, B - NUM_CLASSES), jnp.float32)], axis=1)


def _tc_call(x, cache_keys, W, b):
    return pl.pallas_call(
        _tc_body,
        grid=(NBLK,),
        in_specs=[
            pl.BlockSpec((B, D), lambda i: (0, 0)),
            pl.BlockSpec((BK, D), lambda i: (i, 0)),
            pl.BlockSpec((D, NUM_CLASSES), lambda i: (0, 0)),
            pl.BlockSpec((NUM_CLASSES,), lambda i: (0,)),
        ],
        out_specs=[
            pl.BlockSpec((1, 1, B), lambda i: (i, 0, 0)),
            pl.BlockSpec((B, B), lambda i: (0, 0)),
        ],
        out_shape=[
            jax.ShapeDtypeStruct((NBLK, 1, B), jnp.int32),
            jax.ShapeDtypeStruct((B, B), jnp.float32),
        ],
        compiler_params=pltpu.CompilerParams(
            dimension_semantics=("arbitrary",)),
    )(x, cache_keys, W, b)


def _sc_kernel_fn(cand_hbm, idx_hbm, cache_hbm, cv, idx_v, icv):
    cid = lax.axis_index("c")
    sid = lax.axis_index("s")

    @pl.when(jnp.logical_and(cid == 0, sid == 0))
    def _():
        pltpu.sync_copy(cand_hbm, cv)
        best_p = cv[0, :]
        best_r = jnp.zeros((B,), jnp.int32)
        for r in range(1, NBLK):
            v = cv[r, :]
            m = v < best_p
            best_p = jnp.where(m, v, best_p)
            best_r = jnp.where(m, jnp.full((B,), r, jnp.int32), best_r)
        best_i = (best_p & IDX_MASK) + best_r * BK
        is_cache = (best_p & ~IDX_MASK) <= jnp.int32(0x3F800000)
        idx_v[...] = best_i
        icv[...] = jnp.where(is_cache, jnp.int32(1), jnp.int32(0))
        pltpu.sync_copy(idx_v, idx_hbm)
        pltpu.sync_copy(icv, cache_hbm)


def _sc_call(cand):
    mesh = plsc.VectorSubcoreMesh(core_axis_name="c", subcore_axis_name="s")
    f = functools.partial(
        pl.kernel, mesh=mesh,
        out_type=[
            jax.ShapeDtypeStruct((B,), jnp.int32),
            jax.ShapeDtypeStruct((B,), jnp.int32),
        ],
        scratch_types=[
            pltpu.VMEM((NBLK, B), jnp.int32),
            pltpu.VMEM((B,), jnp.int32),
            pltpu.VMEM((B,), jnp.int32),
        ],
    )(_sc_kernel_fn)
    return f(cand)


def _tc2_body(idx_ref, preds_ref, logits_ref, ic_ref, probs_ref,
              rows_ref, sem):
    copies = [
        pltpu.make_async_copy(
            preds_ref.at[pl.ds(idx_ref[q], 1), :],
            rows_ref.at[pl.ds(q, 1), :], sem)
        for q in range(B)
    ]
    for cp in copies:
        cp.start()
    for cp in copies:
        cp.wait()
    ic = ic_ref[...]
    cached = rows_ref[...]
    model = logits_ref[...][:, :NUM_CLASSES]
    logits = jnp.where(ic > 0, cached, model)
    mx = jnp.max(logits, axis=1, keepdims=True)
    e = jnp.exp(logits - mx)
    probs_ref[...] = e / jnp.sum(e, axis=1, keepdims=True)


def _tc2_call(best_i, cache_preds, logits_pad, ic2d):
    return pl.pallas_call(
        _tc2_body,
        in_specs=[
            pl.BlockSpec(memory_space=pltpu.MemorySpace.SMEM),
            pl.BlockSpec(memory_space=pltpu.MemorySpace.HBM),
            pl.BlockSpec((B, B), lambda: (0, 0)),
            pl.BlockSpec((B, 1), lambda: (0, 0)),
        ],
        out_specs=pl.BlockSpec((B, NUM_CLASSES), lambda: (0, 0)),
        out_shape=jax.ShapeDtypeStruct((B, NUM_CLASSES), jnp.float32),
        scratch_shapes=[
            pltpu.VMEM((B, NUM_CLASSES), jnp.float32),
            pltpu.SemaphoreType.DMA,
        ],
    )(best_i, cache_preds, logits_pad, ic2d)


def kernel(x, cache_keys, cache_preds, W, b):
    cand, logits_pad = _tc_call(x, cache_keys, W, b)
    best_i, is_cache_i32 = _sc_call(cand.reshape(NBLK, B))
    probs = _tc2_call(best_i, cache_preds, logits_pad,
                      is_cache_i32.reshape(B, 1))
    return probs, is_cache_i32.astype(bool)


# speedup vs baseline: 7.5836x; 4.9403x over previous
import functools

import jax
import jax.numpy as jnp
from jax import lax
from jax.experimental import pallas as pl
from jax.experimental.pallas import tpu as pltpu
from jax.experimental.pallas import tpu_sc as plsc

B = 16
D = 64
CACHE_SIZE = 1000000
NUM_CLASSES = 10
THRESHOLD = 1.0

BK = 8192
NBLK = -(-CACHE_SIZE // BK)
IDX_BITS = 13
IDX_MASK = (1 << IDX_BITS) - 1


def _tc_body(x_ref, keyst_ref, wt_ref, b_ref, cand_ref, logits_ref):
    x = x_ref[...]
    kt = keyst_ref[...]
    xsq = jnp.sum(x * x, axis=1, keepdims=True)
    ka = jnp.concatenate([kt, kt * kt], axis=0)
    xa = jnp.concatenate(
        [-2.0 * x, jnp.ones((B, D), jnp.float32)], axis=1)
    s = lax.dot_general(xa.astype(jnp.bfloat16), ka.astype(jnp.bfloat16),
                        (((1,), (0,)), ((), ())),
                        preferred_element_type=jnp.float32)
    d2 = jnp.maximum(s + xsq, 0.0)
    bits = lax.bitcast_convert_type(d2, jnp.int32)
    lane = lax.broadcasted_iota(jnp.int32, (B, BK), 1)
    packed = (bits & ~IDX_MASK) | lane
    limit = CACHE_SIZE - pl.program_id(0) * BK
    packed = jnp.where(lane < limit, packed, jnp.int32(0x7FFFFFFF))
    cand_ref[...] = jnp.min(packed, axis=1).reshape(1, 1, B)

    @pl.when(pl.program_id(0) == 0)
    def _():
        mlt = lax.dot_general(wt_ref[...], x, (((1,), (1,)), ((), ())),
                              preferred_element_type=jnp.float32,
                              precision=lax.Precision.HIGHEST)
        mlt = mlt + b_ref[...][:, None]
        logits_ref[...] = jnp.concatenate(
            [mlt, jnp.zeros((B - NUM_CLASSES, B), jnp.float32)], axis=0)


def _tc_call(x, keys_t, W_t, b):
    return pl.pallas_call(
        _tc_body,
        grid=(NBLK,),
        in_specs=[
            pl.BlockSpec((B, D), lambda i: (0, 0)),
            pl.BlockSpec((D, BK), lambda i: (0, i)),
            pl.BlockSpec((NUM_CLASSES, D), lambda i: (0, 0)),
            pl.BlockSpec((NUM_CLASSES,), lambda i: (0,)),
        ],
        out_specs=[
            pl.BlockSpec((1, 1, B), lambda i: (i, 0, 0)),
            pl.BlockSpec((B, B), lambda i: (0, 0)),
        ],
        out_shape=[
            jax.ShapeDtypeStruct((NBLK, 1, B), jnp.int32),
            jax.ShapeDtypeStruct((B, B), jnp.float32),
        ],
        compiler_params=pltpu.CompilerParams(
            dimension_semantics=("arbitrary",)),
    )(x, keys_t, W_t, b)


def _sc_kernel_fn(cand_hbm, idx_hbm, cache_hbm, cv, idx_v, icv):
    cid = lax.axis_index("c")
    sid = lax.axis_index("s")

    @pl.when(jnp.logical_and(cid == 0, sid == 0))
    def _():
        pltpu.sync_copy(cand_hbm, cv)
        best_p = cv[0, :]
        best_r = jnp.zeros((B,), jnp.int32)
        for r in range(1, NBLK):
            v = cv[r, :]
            m = v < best_p
            best_p = jnp.where(m, v, best_p)
            best_r = jnp.where(m, jnp.full((B,), r, jnp.int32), best_r)
        best_i = (best_p & IDX_MASK) + best_r * BK
        is_cache = (best_p & ~IDX_MASK) <= jnp.int32(0x3F800000)
        idx_v[...] = best_i
        icv[...] = jnp.where(is_cache, jnp.int32(1), jnp.int32(0))
        pltpu.sync_copy(idx_v, idx_hbm)
        pltpu.sync_copy(icv, cache_hbm)


def _sc_call(cand):
    mesh = plsc.VectorSubcoreMesh(core_axis_name="c", subcore_axis_name="s")
    f = functools.partial(
        pl.kernel, mesh=mesh,
        out_type=[
            jax.ShapeDtypeStruct((B,), jnp.int32),
            jax.ShapeDtypeStruct((B,), jnp.int32),
        ],
        scratch_types=[
            pltpu.VMEM((NBLK, B), jnp.int32),
            pltpu.VMEM((B,), jnp.int32),
            pltpu.VMEM((B,), jnp.int32),
        ],
    )(_sc_kernel_fn)
    return f(cand)


def _tc2_body(idx_ref, predst_ref, logits_ref, ic_ref, probs_ref,
              tiles_ref, sem):
    copies = [
        pltpu.make_async_copy(
            predst_ref.at[:, pl.ds(
                pl.multiple_of((idx_ref[q] // 128) * 128, 128), 128)],
            tiles_ref.at[:, pl.ds(q * 128, 128)], sem)
        for q in range(B)
    ]
    for cp in copies:
        cp.start()
    for cp in copies:
        cp.wait()
    lane = lax.broadcasted_iota(jnp.int32, (NUM_CLASSES, 128), 1)
    cols = []
    for q in range(B):
        off = idx_ref[q] % 128
        tile = tiles_ref[:, q * 128:(q + 1) * 128]
        sel = jnp.where(lane == off, tile, jnp.float32(0.0))
        cols.append(jnp.sum(sel, axis=1, keepdims=True))
    cached = jnp.concatenate(cols, axis=1)
    ic = ic_ref[...]
    model = logits_ref[...][:NUM_CLASSES, :]
    logits = jnp.where(ic > 0, cached, model)
    mx = jnp.max(logits, axis=0, keepdims=True)
    e = jnp.exp(logits - mx)
    probs_ref[...] = e / jnp.sum(e, axis=0, keepdims=True)


def _tc2_call(best_i, preds_t, logits_pad, ic2d):
    return pl.pallas_call(
        _tc2_body,
        in_specs=[
            pl.BlockSpec(memory_space=pltpu.MemorySpace.SMEM),
            pl.BlockSpec(memory_space=pltpu.MemorySpace.HBM),
            pl.BlockSpec((B, B), lambda: (0, 0)),
            pl.BlockSpec((1, B), lambda: (0, 0)),
        ],
        out_specs=pl.BlockSpec((NUM_CLASSES, B), lambda: (0, 0)),
        out_shape=jax.ShapeDtypeStruct((NUM_CLASSES, B), jnp.float32),
        scratch_shapes=[
            pltpu.VMEM((NUM_CLASSES, B * 128), jnp.float32),
            pltpu.SemaphoreType.DMA,
        ],
    )(best_i, preds_t, logits_pad, ic2d)


def kernel(x, cache_keys, cache_preds, W, b):
    cand, logits_pad = _tc_call(x, cache_keys.T, W.T, b)
    best_i, is_cache_i32 = _sc_call(cand.reshape(NBLK, B))
    probs_t = _tc2_call(best_i, cache_preds.T, logits_pad,
                        is_cache_i32.reshape(1, B))
    return probs_t.T, is_cache_i32.astype(bool)


# speedup vs baseline: 7.9091x; 1.0429x over previous
import functools

import jax
import jax.numpy as jnp
from jax import lax
from jax.experimental import pallas as pl
from jax.experimental.pallas import tpu as pltpu
from jax.experimental.pallas import tpu_sc as plsc

B = 16
D = 64
CACHE_SIZE = 1000000
NUM_CLASSES = 10
THRESHOLD = 1.0

BK = 8192
NBLK = -(-CACHE_SIZE // BK)
IDX_BITS = 13
IDX_MASK = (1 << IDX_BITS) - 1


def _tc_body(x_ref, keyst_ref, wt_ref, b_ref, cand_ref, logits_ref):
    x = x_ref[...]
    kt = keyst_ref[...]
    xsq = jnp.sum(x * x, axis=1, keepdims=True)
    ka = jnp.concatenate([kt, kt * kt], axis=0)
    xa = jnp.concatenate(
        [-2.0 * x, jnp.ones((B, D), jnp.float32)], axis=1)
    s = lax.dot_general(xa.astype(jnp.bfloat16), ka.astype(jnp.bfloat16),
                        (((1,), (0,)), ((), ())),
                        preferred_element_type=jnp.float32)
    d2 = jnp.maximum(s + xsq, 0.0)
    bits = lax.bitcast_convert_type(d2, jnp.int32)
    lane = lax.broadcasted_iota(jnp.int32, (B, BK), 1)
    packed = (bits & ~IDX_MASK) | lane
    limit = CACHE_SIZE - pl.program_id(0) * BK
    packed = jnp.where(lane < limit, packed, jnp.int32(0x7FFFFFFF))
    cand_ref[pl.ds(pl.program_id(0), 1), :] = (
        jnp.min(packed, axis=1).reshape(1, B))

    @pl.when(pl.program_id(0) == 0)
    def _():
        mlt = lax.dot_general(wt_ref[...], x, (((1,), (1,)), ((), ())),
                              preferred_element_type=jnp.float32,
                              precision=lax.Precision.HIGHEST)
        mlt = mlt + b_ref[...][:, None]
        logits_ref[...] = jnp.concatenate(
            [mlt, jnp.zeros((B - NUM_CLASSES, B), jnp.float32)], axis=0)


def _tc_call(x, keys_t, W_t, b):
    return pl.pallas_call(
        _tc_body,
        grid=(NBLK,),
        in_specs=[
            pl.BlockSpec((B, D), lambda i: (0, 0)),
            pl.BlockSpec((D, BK), lambda i: (0, i)),
            pl.BlockSpec((NUM_CLASSES, D), lambda i: (0, 0)),
            pl.BlockSpec((NUM_CLASSES,), lambda i: (0,)),
        ],
        out_specs=[
            pl.BlockSpec((NBLK, B), lambda i: (0, 0)),
            pl.BlockSpec((B, B), lambda i: (0, 0)),
        ],
        out_shape=[
            jax.ShapeDtypeStruct((NBLK, B), jnp.int32),
            jax.ShapeDtypeStruct((B, B), jnp.float32),
        ],
        compiler_params=pltpu.CompilerParams(
            dimension_semantics=("arbitrary",)),
    )(x, keys_t, W_t, b)


def _sc_kernel_fn(cand_hbm, idx_hbm, cache_hbm, cv, idx_v, icv):
    cid = lax.axis_index("c")
    sid = lax.axis_index("s")

    @pl.when(jnp.logical_and(cid == 0, sid == 0))
    def _():
        pltpu.sync_copy(cand_hbm, cv)
        best_p = cv[0, :]
        best_r = jnp.zeros((B,), jnp.int32)
        for r in range(1, NBLK):
            v = cv[r, :]
            m = v < best_p
            best_p = jnp.where(m, v, best_p)
            best_r = jnp.where(m, jnp.full((B,), r, jnp.int32), best_r)
        best_i = (best_p & IDX_MASK) + best_r * BK
        is_cache = (best_p & ~IDX_MASK) <= jnp.int32(0x3F800000)
        idx_v[...] = best_i
        icv[...] = jnp.where(is_cache, jnp.int32(1), jnp.int32(0))
        pltpu.sync_copy(idx_v, idx_hbm)
        pltpu.sync_copy(icv, cache_hbm)


def _sc_call(cand):
    mesh = plsc.VectorSubcoreMesh(core_axis_name="c", subcore_axis_name="s",
                                  num_cores=1)
    f = functools.partial(
        pl.kernel, mesh=mesh,
        out_type=[
            jax.ShapeDtypeStruct((B,), jnp.int32),
            jax.ShapeDtypeStruct((B,), jnp.int32),
        ],
        scratch_types=[
            pltpu.VMEM((NBLK, B), jnp.int32),
            pltpu.VMEM((B,), jnp.int32),
            pltpu.VMEM((B,), jnp.int32),
        ],
    )(_sc_kernel_fn)
    return f(cand)


def _tc2_body(idx_ref, predst_ref, logits_ref, ic_ref, probs_ref,
              tiles_ref, sem):
    copies = [
        pltpu.make_async_copy(
            predst_ref.at[:, pl.ds(
                pl.multiple_of((idx_ref[q] // 128) * 128, 128), 128)],
            tiles_ref.at[:, pl.ds(q * 128, 128)], sem)
        for q in range(B)
    ]
    for cp in copies:
        cp.start()
    for cp in copies:
        cp.wait()
    lane = lax.broadcasted_iota(jnp.int32, (NUM_CLASSES, 128), 1)
    cols = []
    for q in range(B):
        off = idx_ref[q] % 128
        tile = tiles_ref[:, q * 128:(q + 1) * 128]
        sel = jnp.where(lane == off, tile, jnp.float32(0.0))
        cols.append(jnp.sum(sel, axis=1, keepdims=True))
    cached = jnp.concatenate(cols, axis=1)
    ic = ic_ref[...]
    model = logits_ref[...][:NUM_CLASSES, :]
    logits = jnp.where(ic > 0, cached, model)
    mx = jnp.max(logits, axis=0, keepdims=True)
    e = jnp.exp(logits - mx)
    probs_ref[...] = jnp.transpose(
        e / jnp.sum(e, axis=0, keepdims=True))


def _tc2_call(best_i, preds_t, logits_pad, ic2d):
    return pl.pallas_call(
        _tc2_body,
        in_specs=[
            pl.BlockSpec(memory_space=pltpu.MemorySpace.SMEM),
            pl.BlockSpec(memory_space=pltpu.MemorySpace.HBM),
            pl.BlockSpec((B, B), lambda: (0, 0)),
            pl.BlockSpec((1, B), lambda: (0, 0)),
        ],
        out_specs=pl.BlockSpec((B, NUM_CLASSES), lambda: (0, 0)),
        out_shape=jax.ShapeDtypeStruct((B, NUM_CLASSES), jnp.float32),
        scratch_shapes=[
            pltpu.VMEM((NUM_CLASSES, B * 128), jnp.float32),
            pltpu.SemaphoreType.DMA,
        ],
    )(best_i, preds_t, logits_pad, ic2d)


def kernel(x, cache_keys, cache_preds, W, b):
    cand, logits_pad = _tc_call(x, cache_keys.T, W.T, b)
    best_i, is_cache_i32 = _sc_call(cand)
    probs = _tc2_call(best_i, cache_preds.T, logits_pad,
                      is_cache_i32.reshape(1, B))
    return probs, is_cache_i32.astype(bool)


# speedup vs baseline: 10.4438x; 1.3205x over previous
import functools

import jax
import jax.numpy as jnp
from jax import lax
from jax.experimental import pallas as pl
from jax.experimental.pallas import tpu as pltpu
from jax.experimental.pallas import tpu_sc as plsc

B = 16
D = 64
CACHE_SIZE = 1000000
NUM_CLASSES = 10
THRESHOLD = 1.0

BK = 16384
NBLK = -(-CACHE_SIZE // BK)
IDX_BITS = 14
IDX_MASK = (1 << IDX_BITS) - 1


def _tc_body(x_ref, keyst_ref, wt_ref, b_ref, cand_ref, logits_ref):
    x = x_ref[...]
    kt = keyst_ref[...]
    xsq = jnp.sum(x * x, axis=1, keepdims=True)
    ka = jnp.concatenate([kt, kt * kt], axis=0)
    xa = jnp.concatenate(
        [-2.0 * x, jnp.ones((B, D), jnp.float32)], axis=1)
    s = lax.dot_general(xa.astype(jnp.bfloat16), ka.astype(jnp.bfloat16),
                        (((1,), (0,)), ((), ())),
                        preferred_element_type=jnp.float32)
    d2 = jnp.maximum(s + xsq, 0.0)
    bits = lax.bitcast_convert_type(d2, jnp.int32)
    lane = lax.broadcasted_iota(jnp.int32, (B, BK), 1)
    packed = (bits & ~IDX_MASK) | lane
    limit = CACHE_SIZE - pl.program_id(0) * BK
    packed = jnp.where(lane < limit, packed, jnp.int32(0x7FFFFFFF))
    cand_ref[pl.ds(pl.program_id(0), 1), :] = (
        jnp.min(packed, axis=1).reshape(1, B))

    @pl.when(pl.program_id(0) == 0)
    def _():
        mlt = lax.dot_general(wt_ref[...], x, (((1,), (1,)), ((), ())),
                              preferred_element_type=jnp.float32,
                              precision=lax.Precision.HIGHEST)
        mlt = mlt + b_ref[...][:, None]
        logits_ref[...] = jnp.concatenate(
            [mlt, jnp.zeros((B - NUM_CLASSES, B), jnp.float32)], axis=0)


def _tc_call(x, keys_t, W_t, b):
    return pl.pallas_call(
        _tc_body,
        grid=(NBLK,),
        in_specs=[
            pl.BlockSpec((B, D), lambda i: (0, 0)),
            pl.BlockSpec((D, BK), lambda i: (0, i)),
            pl.BlockSpec((NUM_CLASSES, D), lambda i: (0, 0)),
            pl.BlockSpec((NUM_CLASSES,), lambda i: (0,)),
        ],
        out_specs=[
            pl.BlockSpec((NBLK, B), lambda i: (0, 0)),
            pl.BlockSpec((B, B), lambda i: (0, 0)),
        ],
        out_shape=[
            jax.ShapeDtypeStruct((NBLK, B), jnp.int32),
            jax.ShapeDtypeStruct((B, B), jnp.float32),
        ],
        compiler_params=pltpu.CompilerParams(
            dimension_semantics=("arbitrary",)),
    )(x, keys_t, W_t, b)


def _sc_kernel_fn(cand_hbm, idx_hbm, cache_hbm, cv, idx_v, icv):
    cid = lax.axis_index("c")
    sid = lax.axis_index("s")

    @pl.when(jnp.logical_and(cid == 0, sid == 0))
    def _():
        pltpu.sync_copy(cand_hbm, cv)
        best_p = cv[0, :]
        best_r = jnp.zeros((B,), jnp.int32)
        for r in range(1, NBLK):
            v = cv[r, :]
            m = v < best_p
            best_p = jnp.where(m, v, best_p)
            best_r = jnp.where(m, jnp.full((B,), r, jnp.int32), best_r)
        best_i = (best_p & IDX_MASK) + best_r * BK
        is_cache = (best_p & ~IDX_MASK) <= jnp.int32(0x3F800000)
        idx_v[...] = best_i
        icv[...] = jnp.where(is_cache, jnp.int32(1), jnp.int32(0))
        pltpu.sync_copy(idx_v, idx_hbm)
        pltpu.sync_copy(icv, cache_hbm)


def _sc_call(cand):
    mesh = plsc.VectorSubcoreMesh(core_axis_name="c", subcore_axis_name="s",
                                  num_cores=1)
    f = functools.partial(
        pl.kernel, mesh=mesh,
        out_type=[
            jax.ShapeDtypeStruct((B,), jnp.int32),
            jax.ShapeDtypeStruct((B,), jnp.int32),
        ],
        scratch_types=[
            pltpu.VMEM((NBLK, B), jnp.int32),
            pltpu.VMEM((B,), jnp.int32),
            pltpu.VMEM((B,), jnp.int32),
        ],
    )(_sc_kernel_fn)
    return f(cand)


def _tc2_body(idx_ref, predst_ref, logits_ref, ic_ref, probs_ref,
              tiles_ref, sem):
    copies = [
        pltpu.make_async_copy(
            predst_ref.at[:, pl.ds(
                pl.multiple_of((idx_ref[q] // 128) * 128, 128), 128)],
            tiles_ref.at[:, pl.ds(q * 128, 128)], sem)
        for q in range(B)
    ]
    for cp in copies:
        cp.start()
    for cp in copies:
        cp.wait()
    lane = lax.broadcasted_iota(jnp.int32, (NUM_CLASSES, 128), 1)
    cols = []
    for q in range(B):
        off = idx_ref[q] % 128
        tile = tiles_ref[:, q * 128:(q + 1) * 128]
        sel = jnp.where(lane == off, tile, jnp.float32(0.0))
        cols.append(jnp.sum(sel, axis=1, keepdims=True))
    cached = jnp.concatenate(cols, axis=1)
    ic = ic_ref[...]
    model = logits_ref[...][:NUM_CLASSES, :]
    logits = jnp.where(ic > 0, cached, model)
    mx = jnp.max(logits, axis=0, keepdims=True)
    e = jnp.exp(logits - mx)
    probs_ref[...] = jnp.transpose(
        e / jnp.sum(e, axis=0, keepdims=True))


def _tc2_call(best_i, preds_t, logits_pad, ic2d):
    return pl.pallas_call(
        _tc2_body,
        in_specs=[
            pl.BlockSpec(memory_space=pltpu.MemorySpace.SMEM),
            pl.BlockSpec(memory_space=pltpu.MemorySpace.HBM),
            pl.BlockSpec((B, B), lambda: (0, 0)),
            pl.BlockSpec((1, B), lambda: (0, 0)),
        ],
        out_specs=pl.BlockSpec((B, NUM_CLASSES), lambda: (0, 0)),
        out_shape=jax.ShapeDtypeStruct((B, NUM_CLASSES), jnp.float32),
        scratch_shapes=[
            pltpu.VMEM((NUM_CLASSES, B * 128), jnp.float32),
            pltpu.SemaphoreType.DMA,
        ],
    )(best_i, preds_t, logits_pad, ic2d)


def kernel(x, cache_keys, cache_preds, W, b):
    cand, logits_pad = _tc_call(x, cache_keys.T, W.T, b)
    best_i, is_cache_i32 = _sc_call(cand)
    probs = _tc2_call(best_i, cache_preds.T, logits_pad,
                      is_cache_i32.reshape(1, B))
    return probs, is_cache_i32.astype(bool)


# speedup vs baseline: 12.5009x; 1.1970x over previous
import functools

import jax
import jax.numpy as jnp
from jax import lax
from jax.experimental import pallas as pl
from jax.experimental.pallas import tpu as pltpu
from jax.experimental.pallas import tpu_sc as plsc

B = 16
D = 64
CACHE_SIZE = 1000000
NUM_CLASSES = 10
THRESHOLD = 1.0

BK = 32768
NBLK = -(-CACHE_SIZE // BK)
IDX_BITS = 15
IDX_MASK = (1 << IDX_BITS) - 1


def _tc_body(x_ref, keyst_ref, wt_ref, b_ref, cand_ref, logits_ref):
    x = x_ref[...]
    kt = keyst_ref[...]
    xsq = jnp.sum(x * x, axis=1, keepdims=True)
    ka = jnp.concatenate([kt, kt * kt], axis=0)
    xa = jnp.concatenate(
        [-2.0 * x, jnp.ones((B, D), jnp.float32)], axis=1)
    s = lax.dot_general(xa.astype(jnp.bfloat16), ka.astype(jnp.bfloat16),
                        (((1,), (0,)), ((), ())),
                        preferred_element_type=jnp.float32)
    d2 = jnp.maximum(s + xsq, 0.0)
    bits = lax.bitcast_convert_type(d2, jnp.int32)
    lane = lax.broadcasted_iota(jnp.int32, (B, BK), 1)
    packed = (bits & ~IDX_MASK) | lane
    limit = CACHE_SIZE - pl.program_id(0) * BK
    packed = jnp.where(lane < limit, packed, jnp.int32(0x7FFFFFFF))
    cand_ref[pl.ds(pl.program_id(0), 1), :] = (
        jnp.min(packed, axis=1).reshape(1, B))

    @pl.when(pl.program_id(0) == 0)
    def _():
        mlt = lax.dot_general(wt_ref[...], x, (((1,), (1,)), ((), ())),
                              preferred_element_type=jnp.float32,
                              precision=lax.Precision.HIGHEST)
        mlt = mlt + b_ref[...][:, None]
        logits_ref[...] = jnp.concatenate(
            [mlt, jnp.zeros((B - NUM_CLASSES, B), jnp.float32)], axis=0)


def _tc_call(x, keys_t, W_t, b):
    return pl.pallas_call(
        _tc_body,
        grid=(NBLK,),
        in_specs=[
            pl.BlockSpec((B, D), lambda i: (0, 0)),
            pl.BlockSpec((D, BK), lambda i: (0, i)),
            pl.BlockSpec((NUM_CLASSES, D), lambda i: (0, 0)),
            pl.BlockSpec((NUM_CLASSES,), lambda i: (0,)),
        ],
        out_specs=[
            pl.BlockSpec((NBLK, B), lambda i: (0, 0)),
            pl.BlockSpec((B, B), lambda i: (0, 0)),
        ],
        out_shape=[
            jax.ShapeDtypeStruct((NBLK, B), jnp.int32),
            jax.ShapeDtypeStruct((B, B), jnp.float32),
        ],
        compiler_params=pltpu.CompilerParams(
            dimension_semantics=("arbitrary",)),
    )(x, keys_t, W_t, b)


def _sc_kernel_fn(cand_hbm, idx_hbm, cache_hbm, cv, idx_v, icv):
    cid = lax.axis_index("c")
    sid = lax.axis_index("s")

    @pl.when(jnp.logical_and(cid == 0, sid == 0))
    def _():
        pltpu.sync_copy(cand_hbm, cv)
        best_p = cv[0, :]
        best_r = jnp.zeros((B,), jnp.int32)
        for r in range(1, NBLK):
            v = cv[r, :]
            m = v < best_p
            best_p = jnp.where(m, v, best_p)
            best_r = jnp.where(m, jnp.full((B,), r, jnp.int32), best_r)
        best_i = (best_p & IDX_MASK) + best_r * BK
        is_cache = (best_p & ~IDX_MASK) <= jnp.int32(0x3F800000)
        idx_v[...] = best_i
        icv[...] = jnp.where(is_cache, jnp.int32(1), jnp.int32(0))
        pltpu.sync_copy(idx_v, idx_hbm)
        pltpu.sync_copy(icv, cache_hbm)


def _sc_call(cand):
    mesh = plsc.VectorSubcoreMesh(core_axis_name="c", subcore_axis_name="s",
                                  num_cores=1)
    f = functools.partial(
        pl.kernel, mesh=mesh,
        out_type=[
            jax.ShapeDtypeStruct((B,), jnp.int32),
            jax.ShapeDtypeStruct((B,), jnp.int32),
        ],
        scratch_types=[
            pltpu.VMEM((NBLK, B), jnp.int32),
            pltpu.VMEM((B,), jnp.int32),
            pltpu.VMEM((B,), jnp.int32),
        ],
    )(_sc_kernel_fn)
    return f(cand)


def _tc2_body(idx_ref, predst_ref, logits_ref, ic_ref, probs_ref,
              tiles_ref, sem):
    copies = [
        pltpu.make_async_copy(
            predst_ref.at[:, pl.ds(
                pl.multiple_of((idx_ref[q] // 128) * 128, 128), 128)],
            tiles_ref.at[:, pl.ds(q * 128, 128)], sem)
        for q in range(B)
    ]
    for cp in copies:
        cp.start()
    for cp in copies:
        cp.wait()
    lane = lax.broadcasted_iota(jnp.int32, (NUM_CLASSES, 128), 1)
    cols = []
    for q in range(B):
        off = idx_ref[q] % 128
        tile = tiles_ref[:, q * 128:(q + 1) * 128]
        sel = jnp.where(lane == off, tile, jnp.float32(0.0))
        cols.append(jnp.sum(sel, axis=1, keepdims=True))
    cached = jnp.concatenate(cols, axis=1)
    ic = ic_ref[...]
    model = logits_ref[...][:NUM_CLASSES, :]
    logits = jnp.where(ic > 0, cached, model)
    mx = jnp.max(logits, axis=0, keepdims=True)
    e = jnp.exp(logits - mx)
    probs_ref[...] = jnp.transpose(
        e / jnp.sum(e, axis=0, keepdims=True))


def _tc2_call(best_i, preds_t, logits_pad, ic2d):
    return pl.pallas_call(
        _tc2_body,
        in_specs=[
            pl.BlockSpec(memory_space=pltpu.MemorySpace.SMEM),
            pl.BlockSpec(memory_space=pltpu.MemorySpace.HBM),
            pl.BlockSpec((B, B), lambda: (0, 0)),
            pl.BlockSpec((1, B), lambda: (0, 0)),
        ],
        out_specs=pl.BlockSpec((B, NUM_CLASSES), lambda: (0, 0)),
        out_shape=jax.ShapeDtypeStruct((B, NUM_CLASSES), jnp.float32),
        scratch_shapes=[
            pltpu.VMEM((NUM_CLASSES, B * 128), jnp.float32),
            pltpu.SemaphoreType.DMA,
        ],
    )(best_i, preds_t, logits_pad, ic2d)


def kernel(x, cache_keys, cache_preds, W, b):
    cand, logits_pad = _tc_call(x, cache_keys.T, W.T, b)
    best_i, is_cache_i32 = _sc_call(cand)
    probs = _tc2_call(best_i, cache_preds.T, logits_pad,
                      is_cache_i32.reshape(1, B))
    return probs, is_cache_i32.astype(bool)


# speedup vs baseline: 12.5291x; 1.0023x over previous
import functools

import jax
import jax.numpy as jnp
from jax import lax
from jax.experimental import pallas as pl
from jax.experimental.pallas import tpu as pltpu
from jax.experimental.pallas import tpu_sc as plsc

B = 16
D = 64
CACHE_SIZE = 1000000
NUM_CLASSES = 10
THRESHOLD = 1.0

BK = 65536
NBLK = -(-CACHE_SIZE // BK)
IDX_BITS = 16
IDX_MASK = (1 << IDX_BITS) - 1


def _tc_body(x_ref, keyst_ref, wt_ref, b_ref, cand_ref, logits_ref):
    x = x_ref[...]
    kt = keyst_ref[...]
    xsq = jnp.sum(x * x, axis=1, keepdims=True)
    ka = jnp.concatenate([kt, kt * kt], axis=0)
    xa = jnp.concatenate(
        [-2.0 * x, jnp.ones((B, D), jnp.float32)], axis=1)
    s = lax.dot_general(xa.astype(jnp.bfloat16), ka.astype(jnp.bfloat16),
                        (((1,), (0,)), ((), ())),
                        preferred_element_type=jnp.float32)
    d2 = jnp.maximum(s + xsq, 0.0)
    bits = lax.bitcast_convert_type(d2, jnp.int32)
    lane = lax.broadcasted_iota(jnp.int32, (B, BK), 1)
    packed = (bits & ~IDX_MASK) | lane
    limit = CACHE_SIZE - pl.program_id(0) * BK
    packed = jnp.where(lane < limit, packed, jnp.int32(0x7FFFFFFF))
    cand_ref[pl.ds(pl.program_id(0), 1), :] = (
        jnp.min(packed, axis=1).reshape(1, B))

    @pl.when(pl.program_id(0) == 0)
    def _():
        mlt = lax.dot_general(wt_ref[...], x, (((1,), (1,)), ((), ())),
                              preferred_element_type=jnp.float32,
                              precision=lax.Precision.HIGHEST)
        mlt = mlt + b_ref[...][:, None]
        logits_ref[...] = jnp.concatenate(
            [mlt, jnp.zeros((B - NUM_CLASSES, B), jnp.float32)], axis=0)


def _tc_call(x, keys_t, W_t, b):
    return pl.pallas_call(
        _tc_body,
        grid=(NBLK,),
        in_specs=[
            pl.BlockSpec((B, D), lambda i: (0, 0)),
            pl.BlockSpec((D, BK), lambda i: (0, i)),
            pl.BlockSpec((NUM_CLASSES, D), lambda i: (0, 0)),
            pl.BlockSpec((NUM_CLASSES,), lambda i: (0,)),
        ],
        out_specs=[
            pl.BlockSpec((NBLK, B), lambda i: (0, 0)),
            pl.BlockSpec((B, B), lambda i: (0, 0)),
        ],
        out_shape=[
            jax.ShapeDtypeStruct((NBLK, B), jnp.int32),
            jax.ShapeDtypeStruct((B, B), jnp.float32),
        ],
        compiler_params=pltpu.CompilerParams(
            dimension_semantics=("arbitrary",)),
    )(x, keys_t, W_t, b)


def _sc_kernel_fn(cand_hbm, idx_hbm, cache_hbm, cv, idx_v, icv):
    cid = lax.axis_index("c")
    sid = lax.axis_index("s")

    @pl.when(jnp.logical_and(cid == 0, sid == 0))
    def _():
        pltpu.sync_copy(cand_hbm, cv)
        best_p = cv[0, :]
        best_r = jnp.zeros((B,), jnp.int32)
        for r in range(1, NBLK):
            v = cv[r, :]
            m = v < best_p
            best_p = jnp.where(m, v, best_p)
            best_r = jnp.where(m, jnp.full((B,), r, jnp.int32), best_r)
        best_i = (best_p & IDX_MASK) + best_r * BK
        is_cache = (best_p & ~IDX_MASK) <= jnp.int32(0x3F800000)
        idx_v[...] = best_i
        icv[...] = jnp.where(is_cache, jnp.int32(1), jnp.int32(0))
        pltpu.sync_copy(idx_v, idx_hbm)
        pltpu.sync_copy(icv, cache_hbm)


def _sc_call(cand):
    mesh = plsc.VectorSubcoreMesh(core_axis_name="c", subcore_axis_name="s",
                                  num_cores=1)
    f = functools.partial(
        pl.kernel, mesh=mesh,
        out_type=[
            jax.ShapeDtypeStruct((B,), jnp.int32),
            jax.ShapeDtypeStruct((B,), jnp.int32),
        ],
        scratch_types=[
            pltpu.VMEM((NBLK, B), jnp.int32),
            pltpu.VMEM((B,), jnp.int32),
            pltpu.VMEM((B,), jnp.int32),
        ],
    )(_sc_kernel_fn)
    return f(cand)


def _tc2_body(idx_ref, predst_ref, logits_ref, ic_ref, probs_ref,
              tiles_ref, sem):
    copies = [
        pltpu.make_async_copy(
            predst_ref.at[:, pl.ds(
                pl.multiple_of((idx_ref[q] // 128) * 128, 128), 128)],
            tiles_ref.at[:, pl.ds(q * 128, 128)], sem)
        for q in range(B)
    ]
    for cp in copies:
        cp.start()
    for cp in copies:
        cp.wait()
    lane = lax.broadcasted_iota(jnp.int32, (NUM_CLASSES, 128), 1)
    cols = []
    for q in range(B):
        off = idx_ref[q] % 128
        tile = tiles_ref[:, q * 128:(q + 1) * 128]
        sel = jnp.where(lane == off, tile, jnp.float32(0.0))
        cols.append(jnp.sum(sel, axis=1, keepdims=True))
    cached = jnp.concatenate(cols, axis=1)
    ic = ic_ref[...]
    model = logits_ref[...][:NUM_CLASSES, :]
    logits = jnp.where(ic > 0, cached, model)
    mx = jnp.max(logits, axis=0, keepdims=True)
    e = jnp.exp(logits - mx)
    probs_ref[...] = jnp.transpose(
        e / jnp.sum(e, axis=0, keepdims=True))


def _tc2_call(best_i, preds_t, logits_pad, ic2d):
    return pl.pallas_call(
        _tc2_body,
        in_specs=[
            pl.BlockSpec(memory_space=pltpu.MemorySpace.SMEM),
            pl.BlockSpec(memory_space=pltpu.MemorySpace.HBM),
            pl.BlockSpec((B, B), lambda: (0, 0)),
            pl.BlockSpec((1, B), lambda: (0, 0)),
        ],
        out_specs=pl.BlockSpec((B, NUM_CLASSES), lambda: (0, 0)),
        out_shape=jax.ShapeDtypeStruct((B, NUM_CLASSES), jnp.float32),
        scratch_shapes=[
            pltpu.VMEM((NUM_CLASSES, B * 128), jnp.float32),
            pltpu.SemaphoreType.DMA,
        ],
    )(best_i, preds_t, logits_pad, ic2d)


def kernel(x, cache_keys, cache_preds, W, b):
    cand, logits_pad = _tc_call(x, cache_keys.T, W.T, b)
    best_i, is_cache_i32 = _sc_call(cand)
    probs = _tc2_call(best_i, cache_preds.T, logits_pad,
                      is_cache_i32.reshape(1, B))
    return probs, is_cache_i32.astype(bool)
